# baseline (device time: 106193 ns/iter reference)
import functools

import jax
import jax.numpy as jnp
from jax import lax
from jax.experimental import pallas as pl
from jax.experimental.pallas import tpu as pltpu

N_DEV = 8


def kernel(x, w_mat, scale_x, scale_w):
    m_rows, k_per = x.shape
    _, n = w_mat.shape
    assert m_rows == N_DEV * k_per

    def body(x_ref, w_ref, sx_ref, sw_ref, out_ref,
             x8_ref, w8_ref, x8_recv, w8_recv,
             send_w_sems, send_x_sems, recv_w_sems, recv_x_sems):
        me = lax.axis_index("i")

        barrier_sem = pltpu.get_barrier_semaphore()
        for k in range(1, N_DEV):
            peer = (me + k) % N_DEV
            pl.semaphore_signal(
                barrier_sem, inc=1,
                device_id=(peer,), device_id_type=pl.DeviceIdType.MESH,
            )
        pl.semaphore_wait(barrier_sem, N_DEV - 1)

        x8_ref[...] = x_ref[...].astype(jnp.float8_e5m2)
        w8_ref[...] = w_ref[...].astype(jnp.float8_e5m2)

        sends = []
        for k in range(1, N_DEV):
            dst = (me + k) % N_DEV
            w_rdma = pltpu.make_async_remote_copy(
                src_ref=w8_ref,
                dst_ref=w8_recv.at[me],
                send_sem=send_w_sems.at[k],
                recv_sem=recv_w_sems.at[me],
                device_id=(dst,),
                device_id_type=pl.DeviceIdType.MESH,
            )
            w_rdma.start()
            x_rdma = pltpu.make_async_remote_copy(
                src_ref=x8_ref.at[pl.ds(dst * k_per, k_per), :],
                dst_ref=x8_recv.at[me],
                send_sem=send_x_sems.at[k],
                recv_sem=recv_x_sems.at[me],
                device_id=(dst,),
                device_id_type=pl.DeviceIdType.MESH,
            )
            x_rdma.start()
            sends.append((w_rdma, x_rdma))

        x_own = x8_ref[pl.ds(me * k_per, k_per), :].astype(jnp.bfloat16)
        w_own = w8_ref[...].astype(jnp.bfloat16)
        acc = jnp.dot(x_own, w_own, preferred_element_type=jnp.float32)

        for k in range(1, N_DEV):
            o = (me + k) % N_DEV
            w_wait = pltpu.make_async_remote_copy(
                src_ref=w8_ref,
                dst_ref=w8_recv.at[o],
                send_sem=send_w_sems.at[k],
                recv_sem=recv_w_sems.at[o],
                device_id=(o,),
                device_id_type=pl.DeviceIdType.MESH,
            )
            w_wait.wait_recv()
            x_wait = pltpu.make_async_remote_copy(
                src_ref=x8_ref.at[pl.ds(0, k_per), :],
                dst_ref=x8_recv.at[o],
                send_sem=send_x_sems.at[k],
                recv_sem=recv_x_sems.at[o],
                device_id=(o,),
                device_id_type=pl.DeviceIdType.MESH,
            )
            x_wait.wait_recv()
            xb = x8_recv[o].astype(jnp.bfloat16)
            wb = w8_recv[o].astype(jnp.bfloat16)
            acc = acc + jnp.dot(xb, wb, preferred_element_type=jnp.float32)

        out_ref[...] = acc * (sx_ref[0] * sw_ref[0])

        for w_rdma, x_rdma in sends:
            w_rdma.wait_send()
            x_rdma.wait_send()

    return pl.pallas_call(
        body,
        out_shape=jax.ShapeDtypeStruct((k_per, n), jnp.float32),
        in_specs=[
            pl.BlockSpec(memory_space=pltpu.VMEM),
            pl.BlockSpec(memory_space=pltpu.VMEM),
            pl.BlockSpec(memory_space=pltpu.SMEM),
            pl.BlockSpec(memory_space=pltpu.SMEM),
        ],
        out_specs=pl.BlockSpec(memory_space=pltpu.VMEM),
        scratch_shapes=[
            pltpu.VMEM((m_rows, k_per), jnp.float8_e5m2),
            pltpu.VMEM((k_per, n), jnp.float8_e5m2),
            pltpu.VMEM((N_DEV, k_per, k_per), jnp.float8_e5m2),
            pltpu.VMEM((N_DEV, k_per, n), jnp.float8_e5m2),
            pltpu.SemaphoreType.DMA((N_DEV,)),
            pltpu.SemaphoreType.DMA((N_DEV,)),
            pltpu.SemaphoreType.DMA((N_DEV,)),
            pltpu.SemaphoreType.DMA((N_DEV,)),
        ],
        compiler_params=pltpu.CompilerParams(collective_id=0),
    )(x, w_mat, scale_x, scale_w)


# device time: 98977 ns/iter; 1.0729x vs baseline; 1.0729x over previous
import functools

import jax
import jax.numpy as jnp
from jax import lax
from jax.experimental import pallas as pl
from jax.experimental.pallas import tpu as pltpu

N_DEV = 8


def kernel(x, w_mat, scale_x, scale_w):
    m_rows, k_per = x.shape
    _, n = w_mat.shape
    assert m_rows == N_DEV * k_per

    def body(x_ref, w_ref, sx_ref, sw_ref, out_ref,
             x8_ref, w8_ref, x8_recv, w8_recv,
             send_w_sems, send_x_sems, recv_w_sems, recv_x_sems):
        me = lax.axis_index("i")

        x8_ref[...] = x_ref[...].astype(jnp.float8_e5m2)
        w8_ref[...] = w_ref[...].astype(jnp.float8_e5m2)

        barrier_sem = pltpu.get_barrier_semaphore()
        for k in range(1, N_DEV):
            peer = (me + k) % N_DEV
            pl.semaphore_signal(
                barrier_sem, inc=1,
                device_id=(peer,), device_id_type=pl.DeviceIdType.MESH,
            )
        pl.semaphore_wait(barrier_sem, N_DEV - 1)

        sends = []
        for k in range(1, N_DEV):
            dst = (me + k) % N_DEV
            w_rdma = pltpu.make_async_remote_copy(
                src_ref=w8_ref,
                dst_ref=w8_recv.at[me],
                send_sem=send_w_sems.at[k],
                recv_sem=recv_w_sems.at[me],
                device_id=(dst,),
                device_id_type=pl.DeviceIdType.MESH,
            )
            w_rdma.start()
            x_rdma = pltpu.make_async_remote_copy(
                src_ref=x8_ref.at[pl.ds(dst * k_per, k_per), :],
                dst_ref=x8_recv.at[me],
                send_sem=send_x_sems.at[k],
                recv_sem=recv_x_sems.at[me],
                device_id=(dst,),
                device_id_type=pl.DeviceIdType.MESH,
            )
            x_rdma.start()
            sends.append((w_rdma, x_rdma))

        x_own = x8_ref[pl.ds(me * k_per, k_per), :].astype(jnp.bfloat16)
        w_own = w8_ref[...].astype(jnp.bfloat16)
        acc = jnp.dot(x_own, w_own, preferred_element_type=jnp.float32)

        for k in range(1, N_DEV):
            o = (me - k) % N_DEV
            w_wait = pltpu.make_async_remote_copy(
                src_ref=w8_ref,
                dst_ref=w8_recv.at[o],
                send_sem=send_w_sems.at[k],
                recv_sem=recv_w_sems.at[o],
                device_id=(o,),
                device_id_type=pl.DeviceIdType.MESH,
            )
            w_wait.wait_recv()
            x_wait = pltpu.make_async_remote_copy(
                src_ref=x8_ref.at[pl.ds(0, k_per), :],
                dst_ref=x8_recv.at[o],
                send_sem=send_x_sems.at[k],
                recv_sem=recv_x_sems.at[o],
                device_id=(o,),
                device_id_type=pl.DeviceIdType.MESH,
            )
            x_wait.wait_recv()
            xb = x8_recv[o].astype(jnp.bfloat16)
            wb = w8_recv[o].astype(jnp.bfloat16)
            acc = acc + jnp.dot(xb, wb, preferred_element_type=jnp.float32)

        out_ref[...] = acc * (sx_ref[0] * sw_ref[0])

        for w_rdma, x_rdma in sends:
            w_rdma.wait_send()
            x_rdma.wait_send()

    return pl.pallas_call(
        body,
        out_shape=jax.ShapeDtypeStruct((k_per, n), jnp.float32),
        in_specs=[
            pl.BlockSpec(memory_space=pltpu.VMEM),
            pl.BlockSpec(memory_space=pltpu.VMEM),
            pl.BlockSpec(memory_space=pltpu.SMEM),
            pl.BlockSpec(memory_space=pltpu.SMEM),
        ],
        out_specs=pl.BlockSpec(memory_space=pltpu.VMEM),
        scratch_shapes=[
            pltpu.VMEM((m_rows, k_per), jnp.float8_e5m2),
            pltpu.VMEM((k_per, n), jnp.float8_e5m2),
            pltpu.VMEM((N_DEV, k_per, k_per), jnp.float8_e5m2),
            pltpu.VMEM((N_DEV, k_per, n), jnp.float8_e5m2),
            pltpu.SemaphoreType.DMA((N_DEV,)),
            pltpu.SemaphoreType.DMA((N_DEV,)),
            pltpu.SemaphoreType.DMA((N_DEV,)),
            pltpu.SemaphoreType.DMA((N_DEV,)),
        ],
        compiler_params=pltpu.CompilerParams(collective_id=0),
    )(x, w_mat, scale_x, scale_w)
